# speedup vs baseline: 1.1570x; 1.1570x over previous
import jax
import jax.numpy as jnp
from jax.experimental import pallas as pl

N = 10000
K = 32
D_IN = 128
D_OUT = 128

_BLOCK = 400


def _body(root_ref, nbr_ref, w_ref, out_ref):
    s = jnp.sum(nbr_ref[...], axis=1) + root_ref[...]
    w = w_ref[...] * (1.0 / (K + 1))
    out_ref[...] = jnp.maximum(
        jnp.dot(s, w, preferred_element_type=jnp.float32), 0.0
    )


def kernel(root_feature, neighbor_features, W):
    return pl.pallas_call(
        _body,
        grid=(N // _BLOCK,),
        in_specs=[
            pl.BlockSpec((_BLOCK, D_IN), lambda i: (i, 0)),
            pl.BlockSpec((_BLOCK, K, D_IN), lambda i: (i, 0, 0)),
            pl.BlockSpec((D_IN, D_OUT), lambda i: (0, 0)),
        ],
        out_specs=pl.BlockSpec((_BLOCK, D_OUT), lambda i: (i, 0)),
        out_shape=jax.ShapeDtypeStruct((N, D_OUT), jnp.float32),
    )(root_feature, neighbor_features, W)
